# baseline (speedup 1.0000x reference)
import numpy as np
import jax
import jax.numpy as jnp
from jax.experimental import pallas as pl
from jax.experimental.pallas import tpu as pltpu
from jax.experimental.pallas import tpu_sc as plsc

SEQ = 2048
NF = SEQ // 2
HID = 4096
NPS = 88
NPSP = 96
HBLK = 1024
_SC_LANES = 16
_NEG = float(-3.0e38)

_l = np.arange(SEQ, dtype=np.float64)[:, None]
_k = np.arange(1, NF + 1, dtype=np.float64)[None, :]
_ang = (2.0 * np.pi / SEQ) * _l * _k
_D2_NP = (np.concatenate([np.cos(_ang), -np.sin(_ang)], axis=1)
          / np.sqrt(SEQ)).astype(np.float32)


def _front_body(xt_ref, wext_ref, d2_ref, o_ref, xs_s):
    i = pl.program_id(0)
    nb = pl.num_programs(0)
    xs_s[pl.ds(i, 1), :] = jnp.dot(wext_ref[...], xt_ref[...],
                                   preferred_element_type=jnp.float32)

    @pl.when(i == nb - 1)
    def _fin():
        o_ref[...] = jnp.dot(xs_s[...], d2_ref[...],
                             preferred_element_type=jnp.float32)


def _mlp_body(xri_ref, w1_ref, b1_ref, w2_ref, b2_ref, wg_ref, wn_ref,
              tr_ref, eps_ref, o_ref, accr, acci):
    i = pl.program_id(0)
    nsteps = pl.num_programs(0)
    bsz = xri_ref.shape[0]

    xr = xri_ref[:, :NF]
    xi = xri_ref[:, NF:]
    a2 = jnp.concatenate([xr, xi], axis=0)
    b2m = jnp.concatenate([xi, xr], axis=0)

    y0 = jnp.dot(a2, w1_ref[0], preferred_element_type=jnp.float32)
    y1 = jnp.dot(b2m, w1_ref[1], preferred_element_type=jnp.float32)
    o1r = jax.nn.relu(y0[:bsz] - y1[:bsz] + b1_ref[0])
    o1i = jax.nn.relu(y0[bsz:] + y1[bsz:] + b1_ref[1])
    o2 = jnp.concatenate([o1r, o1i], axis=0)

    u = jnp.dot(o2, w2_ref[0], preferred_element_type=jnp.float32)
    v = jnp.dot(o2, w2_ref[1], preferred_element_type=jnp.float32)

    @pl.when(i == 0)
    def _init():
        accr[...] = jnp.zeros_like(accr)
        acci[...] = jnp.zeros_like(acci)

    accr[...] += u[:bsz] - v[bsz:]
    acci[...] += u[bsz:] + v[:bsz]

    @pl.when(i == nsteps - 1)
    def _fin():
        ar = accr[...] + b2_ref[0]
        ai = acci[...] + b2_ref[1]
        mag = jnp.sqrt(ar * ar + ai * ai)
        clean = jnp.dot(mag, wg_ref[...], preferred_element_type=jnp.float32)
        zn = jnp.dot(mag, wn_ref[...], preferred_element_type=jnp.float32)
        softplus = jnp.log1p(jnp.exp(-jnp.abs(zn))) + jnp.maximum(zn, 0.0)
        noisy = clean + softplus + eps_ref[0, 0]
        logits = jnp.where(tr_ref[0, 0] != 0.0, noisy, clean)
        o_ref[...] = jnp.concatenate(
            [logits, jnp.full((bsz, NPSP - NPS), _NEG, jnp.float32)], axis=1)


def _route_sc_body(lg_hbm, out_hbm, row_v, out_v):
    wid = jax.lax.axis_index("s") * 2 + jax.lax.axis_index("c")
    pltpu.sync_copy(lg_hbm.at[wid], row_v)
    nch = NPSP // _SC_LANES
    big = jnp.int32(4096)

    m1 = jnp.float32(_NEG)
    for t in range(nch):
        v = row_v[pl.ds(t * _SC_LANES, _SC_LANES)]
        m1 = jnp.maximum(m1, jax.lax.reduce_max(v, axes=(0,)))
    i1 = big
    for t in range(nch):
        v = row_v[pl.ds(t * _SC_LANES, _SC_LANES)]
        idx = jax.lax.iota(jnp.int32, _SC_LANES) + jnp.int32(t * _SC_LANES)
        i1 = jnp.minimum(i1, jax.lax.reduce_min(
            jnp.where(v == m1, idx, big), axes=(0,)))
    m2 = jnp.float32(_NEG)
    for t in range(nch):
        v = row_v[pl.ds(t * _SC_LANES, _SC_LANES)]
        idx = jax.lax.iota(jnp.int32, _SC_LANES) + jnp.int32(t * _SC_LANES)
        m2 = jnp.maximum(m2, jax.lax.reduce_max(
            jnp.where(idx == i1, jnp.float32(_NEG), v), axes=(0,)))
    i2 = big
    for t in range(nch):
        v = row_v[pl.ds(t * _SC_LANES, _SC_LANES)]
        idx = jax.lax.iota(jnp.int32, _SC_LANES) + jnp.int32(t * _SC_LANES)
        i2 = jnp.minimum(i2, jax.lax.reduce_min(
            jnp.where((v == m2) & (idx != i1), idx, big), axes=(0,)))

    ev = jnp.exp(jnp.full((_SC_LANES,), m2 - m1, jnp.float32))
    p1 = 1.0 / (1.0 + ev)
    p2 = ev / (1.0 + ev)
    for t in range(nch):
        idx = jax.lax.iota(jnp.int32, _SC_LANES) + jnp.int32(t * _SC_LANES)
        w = (jnp.where(idx == i1, p1, jnp.float32(0.0))
             + jnp.where(idx == i2, p2, jnp.float32(0.0)))
        out_v[pl.ds(t * _SC_LANES, _SC_LANES)] = w
    pltpu.sync_copy(out_v, out_hbm.at[wid])


def kernel(x, start_w, start_b, w1, b1, w2, b2, w_gate, w_noise,
           training=False, noise_epsilon=0.01):
    B_, C_, L_, N_ = x.shape
    CN = C_ * N_
    xt = x.transpose(0, 1, 3, 2).reshape(B_ * CN, L_)
    wext = jnp.repeat(start_w[0] / N_, N_).reshape(1, CN)

    d2 = jnp.asarray(_D2_NP)
    xri = pl.pallas_call(
        _front_body,
        grid=(B_,),
        in_specs=[
            pl.BlockSpec((CN, L_), lambda i: (i, 0)),
            pl.BlockSpec((1, CN), lambda i: (0, 0)),
            pl.BlockSpec((SEQ, 2 * NF), lambda i: (0, 0)),
        ],
        out_specs=pl.BlockSpec((B_, 2 * NF), lambda i: (0, 0)),
        out_shape=jax.ShapeDtypeStruct((B_, 2 * NF), jnp.float32),
        scratch_shapes=[pltpu.VMEM((B_, SEQ), jnp.float32)],
    )(xt, wext, d2)

    tr = jnp.asarray(training, jnp.float32).reshape(1, 1)
    eps = jnp.asarray(noise_epsilon, jnp.float32).reshape(1, 1)
    nh = HID // HBLK

    logits96 = pl.pallas_call(
        _mlp_body,
        grid=(nh,),
        in_specs=[
            pl.BlockSpec((B_, 2 * NF), lambda i: (0, 0)),
            pl.BlockSpec((2, NF, HBLK), lambda i: (0, 0, i)),
            pl.BlockSpec((2, HBLK), lambda i: (0, i)),
            pl.BlockSpec((2, HBLK, NF), lambda i: (0, i, 0)),
            pl.BlockSpec((2, NF), lambda i: (0, 0)),
            pl.BlockSpec((NF, NPS), lambda i: (0, 0)),
            pl.BlockSpec((NF, NPS), lambda i: (0, 0)),
            pl.BlockSpec((1, 1), lambda i: (0, 0),
                         memory_space=pltpu.SMEM),
            pl.BlockSpec((1, 1), lambda i: (0, 0),
                         memory_space=pltpu.SMEM),
        ],
        out_specs=pl.BlockSpec((B_, NPSP), lambda i: (0, 0)),
        out_shape=jax.ShapeDtypeStruct((B_, NPSP), jnp.float32),
        scratch_shapes=[pltpu.VMEM((B_, NF), jnp.float32),
                        pltpu.VMEM((B_, NF), jnp.float32)],
    )(xri, w1, b1, w2, b2, w_gate, w_noise, tr, eps)

    mesh = plsc.VectorSubcoreMesh(core_axis_name="c", subcore_axis_name="s")
    gates96 = pl.kernel(
        _route_sc_body,
        out_type=jax.ShapeDtypeStruct((B_, NPSP), jnp.float32),
        mesh=mesh,
        scratch_types=[pltpu.VMEM((NPSP,), jnp.float32),
                       pltpu.VMEM((NPSP,), jnp.float32)],
        compiler_params=pltpu.CompilerParams(needs_layout_passes=False),
    )(logits96)
    return gates96[:, :NPS]


# speedup vs baseline: 1.0252x; 1.0252x over previous
import numpy as np
import jax
import jax.numpy as jnp
from jax.experimental import pallas as pl
from jax.experimental.pallas import tpu as pltpu
from jax.experimental.pallas import tpu_sc as plsc

SEQ = 2048
NF = SEQ // 2
HID = 4096
NPS = 88
NPSP = 96
HBLK = 1024
_SC_LANES = 16
_NEG = float(-3.0e38)

_l = np.arange(SEQ, dtype=np.float64)[:, None]
_k = np.arange(1, NF + 1, dtype=np.float64)[None, :]
_ang = (2.0 * np.pi / SEQ) * _l * _k
_D2_NP = (np.concatenate([np.cos(_ang), -np.sin(_ang)], axis=1)
          / np.sqrt(SEQ)).astype(np.float32)


def _front_body(xt_ref, wext_ref, d2_ref, o_ref, xs_s):
    i = pl.program_id(0)
    nb = pl.num_programs(0)
    cn = wext_ref.shape[1]
    nrows = xt_ref.shape[0] // cn
    for j in range(nrows):
        xs_s[pl.ds(i * nrows + j, 1), :] = jnp.dot(
            wext_ref[...], xt_ref[pl.ds(j * cn, cn), :],
            preferred_element_type=jnp.float32)

    @pl.when(i == nb - 1)
    def _fin():
        o_ref[...] = jnp.dot(xs_s[...], d2_ref[...],
                             preferred_element_type=jnp.float32)


def _mlp_body(xri_ref, w1_ref, b1_ref, w2_ref, b2_ref, wg_ref, wn_ref,
              tr_ref, eps_ref, o_ref, accr, acci):
    i = pl.program_id(0)
    nsteps = pl.num_programs(0)
    bsz = xri_ref.shape[0]

    xr = xri_ref[:, :NF]
    xi = xri_ref[:, NF:]
    a2 = jnp.concatenate([xr, xi], axis=0)
    b2m = jnp.concatenate([xi, xr], axis=0)

    y0 = jnp.dot(a2, w1_ref[0], preferred_element_type=jnp.float32)
    y1 = jnp.dot(b2m, w1_ref[1], preferred_element_type=jnp.float32)
    o1r = jax.nn.relu(y0[:bsz] - y1[:bsz] + b1_ref[0])
    o1i = jax.nn.relu(y0[bsz:] + y1[bsz:] + b1_ref[1])
    o2 = jnp.concatenate([o1r, o1i], axis=0)

    u = jnp.dot(o2, w2_ref[0], preferred_element_type=jnp.float32)
    v = jnp.dot(o2, w2_ref[1], preferred_element_type=jnp.float32)

    @pl.when(i == 0)
    def _init():
        accr[...] = jnp.zeros_like(accr)
        acci[...] = jnp.zeros_like(acci)

    accr[...] += u[:bsz] - v[bsz:]
    acci[...] += u[bsz:] + v[:bsz]

    @pl.when(i == nsteps - 1)
    def _fin():
        ar = accr[...] + b2_ref[0]
        ai = acci[...] + b2_ref[1]
        mag = jnp.sqrt(ar * ar + ai * ai)
        clean = jnp.dot(mag, wg_ref[...], preferred_element_type=jnp.float32)
        zn = jnp.dot(mag, wn_ref[...], preferred_element_type=jnp.float32)
        softplus = jnp.log1p(jnp.exp(-jnp.abs(zn))) + jnp.maximum(zn, 0.0)
        noisy = clean + softplus + eps_ref[0, 0]
        logits = jnp.where(tr_ref[0, 0] != 0.0, noisy, clean)
        o_ref[...] = jnp.concatenate(
            [logits, jnp.full((bsz, NPSP - NPS), _NEG, jnp.float32)], axis=1)


def _route_sc_body(lg_hbm, out_hbm, row_v, out_v):
    wid = jax.lax.axis_index("s") * 2 + jax.lax.axis_index("c")
    pltpu.sync_copy(lg_hbm.at[wid], row_v)
    nch = NPSP // _SC_LANES
    big = jnp.int32(4096)

    m1 = jnp.float32(_NEG)
    for t in range(nch):
        v = row_v[pl.ds(t * _SC_LANES, _SC_LANES)]
        m1 = jnp.maximum(m1, jax.lax.reduce_max(v, axes=(0,)))
    i1 = big
    for t in range(nch):
        v = row_v[pl.ds(t * _SC_LANES, _SC_LANES)]
        idx = jax.lax.iota(jnp.int32, _SC_LANES) + jnp.int32(t * _SC_LANES)
        i1 = jnp.minimum(i1, jax.lax.reduce_min(
            jnp.where(v == m1, idx, big), axes=(0,)))
    m2 = jnp.float32(_NEG)
    for t in range(nch):
        v = row_v[pl.ds(t * _SC_LANES, _SC_LANES)]
        idx = jax.lax.iota(jnp.int32, _SC_LANES) + jnp.int32(t * _SC_LANES)
        m2 = jnp.maximum(m2, jax.lax.reduce_max(
            jnp.where(idx == i1, jnp.float32(_NEG), v), axes=(0,)))
    i2 = big
    for t in range(nch):
        v = row_v[pl.ds(t * _SC_LANES, _SC_LANES)]
        idx = jax.lax.iota(jnp.int32, _SC_LANES) + jnp.int32(t * _SC_LANES)
        i2 = jnp.minimum(i2, jax.lax.reduce_min(
            jnp.where((v == m2) & (idx != i1), idx, big), axes=(0,)))

    ev = jnp.exp(jnp.full((_SC_LANES,), m2 - m1, jnp.float32))
    p1 = 1.0 / (1.0 + ev)
    p2 = ev / (1.0 + ev)
    for t in range(nch):
        idx = jax.lax.iota(jnp.int32, _SC_LANES) + jnp.int32(t * _SC_LANES)
        w = (jnp.where(idx == i1, p1, jnp.float32(0.0))
             + jnp.where(idx == i2, p2, jnp.float32(0.0)))
        out_v[pl.ds(t * _SC_LANES, _SC_LANES)] = w
    pltpu.sync_copy(out_v, out_hbm.at[wid])


def kernel(x, start_w, start_b, w1, b1, w2, b2, w_gate, w_noise,
           training=False, noise_epsilon=0.01):
    B_, C_, L_, N_ = x.shape
    CN = C_ * N_
    xt = x.transpose(0, 1, 3, 2).reshape(B_ * CN, L_)
    wext = jnp.repeat(start_w[0] / N_, N_).reshape(1, CN)

    d2 = jnp.asarray(_D2_NP)
    BPG = 2
    xri = pl.pallas_call(
        _front_body,
        grid=(B_ // BPG,),
        in_specs=[
            pl.BlockSpec((BPG * CN, L_), lambda i: (i, 0)),
            pl.BlockSpec((1, CN), lambda i: (0, 0)),
            pl.BlockSpec((SEQ, 2 * NF), lambda i: (0, 0)),
        ],
        out_specs=pl.BlockSpec((B_, 2 * NF), lambda i: (0, 0)),
        out_shape=jax.ShapeDtypeStruct((B_, 2 * NF), jnp.float32),
        scratch_shapes=[pltpu.VMEM((B_, SEQ), jnp.float32)],
    )(xt, wext, d2)

    tr = jnp.asarray(training, jnp.float32).reshape(1, 1)
    eps = jnp.asarray(noise_epsilon, jnp.float32).reshape(1, 1)
    nh = HID // HBLK

    logits96 = pl.pallas_call(
        _mlp_body,
        grid=(nh,),
        in_specs=[
            pl.BlockSpec((B_, 2 * NF), lambda i: (0, 0)),
            pl.BlockSpec((2, NF, HBLK), lambda i: (0, 0, i)),
            pl.BlockSpec((2, HBLK), lambda i: (0, i)),
            pl.BlockSpec((2, HBLK, NF), lambda i: (0, i, 0)),
            pl.BlockSpec((2, NF), lambda i: (0, 0)),
            pl.BlockSpec((NF, NPS), lambda i: (0, 0)),
            pl.BlockSpec((NF, NPS), lambda i: (0, 0)),
            pl.BlockSpec((1, 1), lambda i: (0, 0),
                         memory_space=pltpu.SMEM),
            pl.BlockSpec((1, 1), lambda i: (0, 0),
                         memory_space=pltpu.SMEM),
        ],
        out_specs=pl.BlockSpec((B_, NPSP), lambda i: (0, 0)),
        out_shape=jax.ShapeDtypeStruct((B_, NPSP), jnp.float32),
        scratch_shapes=[pltpu.VMEM((B_, NF), jnp.float32),
                        pltpu.VMEM((B_, NF), jnp.float32)],
    )(xri, w1, b1, w2, b2, w_gate, w_noise, tr, eps)

    mesh = plsc.VectorSubcoreMesh(core_axis_name="c", subcore_axis_name="s")
    gates96 = pl.kernel(
        _route_sc_body,
        out_type=jax.ShapeDtypeStruct((B_, NPSP), jnp.float32),
        mesh=mesh,
        scratch_types=[pltpu.VMEM((NPSP,), jnp.float32),
                       pltpu.VMEM((NPSP,), jnp.float32)],
        compiler_params=pltpu.CompilerParams(needs_layout_passes=False),
    )(logits96)
    return gates96[:, :NPS]


# speedup vs baseline: 1.0347x; 1.0092x over previous
import numpy as np
import jax
import jax.numpy as jnp
from jax.experimental import pallas as pl
from jax.experimental.pallas import tpu as pltpu
from jax.experimental.pallas import tpu_sc as plsc

SEQ = 2048
NF = SEQ // 2
HID = 4096
NPS = 88
NPSP = 96
HBLK = 1024
_SC_LANES = 16
_NEG = float(-3.0e38)

_l = np.arange(SEQ, dtype=np.float64)[:, None]
_k = np.arange(1, NF + 1, dtype=np.float64)[None, :]
_ang = (2.0 * np.pi / SEQ) * _l * _k
_D2_NP = (np.concatenate([np.cos(_ang), -np.sin(_ang)], axis=1)
          / np.sqrt(SEQ)).astype(np.float32)


def _front_body(xt_ref, wext_ref, d2_ref, o_ref, xs_s):
    i = pl.program_id(0)
    nb = pl.num_programs(0)
    cn = wext_ref.shape[1]
    nrows = xt_ref.shape[0] // cn
    for j in range(nrows):
        xs_s[pl.ds(i * nrows + j, 1), :] = jnp.dot(
            wext_ref[...], xt_ref[pl.ds(j * cn, cn), :],
            preferred_element_type=jnp.float32)

    @pl.when(i == nb - 1)
    def _fin():
        o_ref[...] = jnp.dot(xs_s[...], d2_ref[...],
                             preferred_element_type=jnp.float32)


def _mlp_body(xri_ref, w1_ref, b1_ref, w2_ref, b2_ref, wg_ref, wn_ref,
              tr_ref, eps_ref, o_ref, accr, acci):
    i = pl.program_id(0)
    nsteps = pl.num_programs(0)
    bsz = xri_ref.shape[0]

    xr = xri_ref[:, :NF]
    xi = xri_ref[:, NF:]
    a2 = jnp.concatenate([xr, xi], axis=0)
    b2m = jnp.concatenate([xi, xr], axis=0)

    y0 = jnp.dot(a2, w1_ref[0], preferred_element_type=jnp.float32)
    y1 = jnp.dot(b2m, w1_ref[1], preferred_element_type=jnp.float32)
    o1r = jax.nn.relu(y0[:bsz] - y1[:bsz] + b1_ref[0])
    o1i = jax.nn.relu(y0[bsz:] + y1[bsz:] + b1_ref[1])
    o2 = jnp.concatenate([o1r, o1i], axis=0)

    u = jnp.dot(o2, w2_ref[0], preferred_element_type=jnp.float32)
    v = jnp.dot(o2, w2_ref[1], preferred_element_type=jnp.float32)

    @pl.when(i == 0)
    def _init():
        accr[...] = jnp.zeros_like(accr)
        acci[...] = jnp.zeros_like(acci)

    accr[...] += u[:bsz] - v[bsz:]
    acci[...] += u[bsz:] + v[:bsz]

    @pl.when(i == nsteps - 1)
    def _fin():
        ar = accr[...] + b2_ref[0]
        ai = acci[...] + b2_ref[1]
        mag = jnp.sqrt(ar * ar + ai * ai)
        clean = jnp.dot(mag, wg_ref[...], preferred_element_type=jnp.float32)
        zn = jnp.dot(mag, wn_ref[...], preferred_element_type=jnp.float32)
        softplus = jnp.log1p(jnp.exp(-jnp.abs(zn))) + jnp.maximum(zn, 0.0)
        noisy = clean + softplus + eps_ref[0, 0]
        logits = jnp.where(tr_ref[0, 0] != 0.0, noisy, clean)
        o_ref[...] = jnp.concatenate(
            [logits, jnp.full((bsz, NPSP - NPS), _NEG, jnp.float32)], axis=1)


def _route_sc_body(lg_hbm, out_hbm, row_v, out_v):
    wid = jax.lax.axis_index("s") * 2 + jax.lax.axis_index("c")
    pltpu.sync_copy(lg_hbm.at[wid], row_v)
    nch = NPSP // _SC_LANES
    big = jnp.int32(4096)

    m1 = jnp.float32(_NEG)
    for t in range(nch):
        v = row_v[pl.ds(t * _SC_LANES, _SC_LANES)]
        m1 = jnp.maximum(m1, jax.lax.reduce_max(v, axes=(0,)))
    i1 = big
    for t in range(nch):
        v = row_v[pl.ds(t * _SC_LANES, _SC_LANES)]
        idx = jax.lax.iota(jnp.int32, _SC_LANES) + jnp.int32(t * _SC_LANES)
        i1 = jnp.minimum(i1, jax.lax.reduce_min(
            jnp.where(v == m1, idx, big), axes=(0,)))
    m2 = jnp.float32(_NEG)
    for t in range(nch):
        v = row_v[pl.ds(t * _SC_LANES, _SC_LANES)]
        idx = jax.lax.iota(jnp.int32, _SC_LANES) + jnp.int32(t * _SC_LANES)
        m2 = jnp.maximum(m2, jax.lax.reduce_max(
            jnp.where(idx == i1, jnp.float32(_NEG), v), axes=(0,)))
    i2 = big
    for t in range(nch):
        v = row_v[pl.ds(t * _SC_LANES, _SC_LANES)]
        idx = jax.lax.iota(jnp.int32, _SC_LANES) + jnp.int32(t * _SC_LANES)
        i2 = jnp.minimum(i2, jax.lax.reduce_min(
            jnp.where((v == m2) & (idx != i1), idx, big), axes=(0,)))

    ev = jnp.exp(jnp.full((_SC_LANES,), m2 - m1, jnp.float32))
    p1 = 1.0 / (1.0 + ev)
    p2 = ev / (1.0 + ev)
    for t in range(nch):
        idx = jax.lax.iota(jnp.int32, _SC_LANES) + jnp.int32(t * _SC_LANES)
        w = (jnp.where(idx == i1, p1, jnp.float32(0.0))
             + jnp.where(idx == i2, p2, jnp.float32(0.0)))
        out_v[pl.ds(t * _SC_LANES, _SC_LANES)] = w
    pltpu.sync_copy(out_v, out_hbm.at[wid])


def kernel(x, start_w, start_b, w1, b1, w2, b2, w_gate, w_noise,
           training=False, noise_epsilon=0.01):
    B_, C_, L_, N_ = x.shape
    CN = C_ * N_
    xt = x.transpose(0, 1, 3, 2).reshape(B_ * CN, L_)
    wext = jnp.repeat(start_w[0] / N_, N_).reshape(1, CN)

    d2 = jnp.asarray(_D2_NP)
    BPG = 4
    xri = pl.pallas_call(
        _front_body,
        grid=(B_ // BPG,),
        in_specs=[
            pl.BlockSpec((BPG * CN, L_), lambda i: (i, 0)),
            pl.BlockSpec((1, CN), lambda i: (0, 0)),
            pl.BlockSpec((SEQ, 2 * NF), lambda i: (0, 0)),
        ],
        out_specs=pl.BlockSpec((B_, 2 * NF), lambda i: (0, 0)),
        out_shape=jax.ShapeDtypeStruct((B_, 2 * NF), jnp.float32),
        scratch_shapes=[pltpu.VMEM((B_, SEQ), jnp.float32)],
    )(xt, wext, d2)

    tr = jnp.asarray(training, jnp.float32).reshape(1, 1)
    eps = jnp.asarray(noise_epsilon, jnp.float32).reshape(1, 1)
    nh = HID // HBLK

    logits96 = pl.pallas_call(
        _mlp_body,
        grid=(nh,),
        in_specs=[
            pl.BlockSpec((B_, 2 * NF), lambda i: (0, 0)),
            pl.BlockSpec((2, NF, HBLK), lambda i: (0, 0, i)),
            pl.BlockSpec((2, HBLK), lambda i: (0, i)),
            pl.BlockSpec((2, HBLK, NF), lambda i: (0, i, 0)),
            pl.BlockSpec((2, NF), lambda i: (0, 0)),
            pl.BlockSpec((NF, NPS), lambda i: (0, 0)),
            pl.BlockSpec((NF, NPS), lambda i: (0, 0)),
            pl.BlockSpec((1, 1), lambda i: (0, 0),
                         memory_space=pltpu.SMEM),
            pl.BlockSpec((1, 1), lambda i: (0, 0),
                         memory_space=pltpu.SMEM),
        ],
        out_specs=pl.BlockSpec((B_, NPSP), lambda i: (0, 0)),
        out_shape=jax.ShapeDtypeStruct((B_, NPSP), jnp.float32),
        scratch_shapes=[pltpu.VMEM((B_, NF), jnp.float32),
                        pltpu.VMEM((B_, NF), jnp.float32)],
    )(xri, w1, b1, w2, b2, w_gate, w_noise, tr, eps)

    mesh = plsc.VectorSubcoreMesh(core_axis_name="c", subcore_axis_name="s")
    gates96 = pl.kernel(
        _route_sc_body,
        out_type=jax.ShapeDtypeStruct((B_, NPSP), jnp.float32),
        mesh=mesh,
        scratch_types=[pltpu.VMEM((NPSP,), jnp.float32),
                       pltpu.VMEM((NPSP,), jnp.float32)],
        compiler_params=pltpu.CompilerParams(needs_layout_passes=False),
    )(logits96)
    return gates96[:, :NPS]
